# baseline (speedup 1.0000x reference)
import functools

import jax
import jax.numpy as jnp
from jax import lax
from jax.experimental import pallas as pl
from jax.experimental.pallas import tpu as pltpu
from jax.experimental.pallas import tpu_sc as plsc

VOCAB = 1000
HIDDEN = 128
BATCH = 1024
SEQ = 50

def _table_body(embed_ref, proj_ref, bias_ref, table_ref):
    table_ref[...] = (
        lax.dot_general(
            embed_ref[...],
            proj_ref[...],
            dimension_numbers=(((1,), (1,)), ((), ())),
            preferred_element_type=jnp.float32,
        )
        + bias_ref[...]
    )


def _build_table(embed_weight, proj_weight, proj_bias):
    return pl.pallas_call(
        _table_body,
        out_shape=jax.ShapeDtypeStruct((VOCAB, VOCAB), jnp.float32),
    )(embed_weight, proj_weight, proj_bias.reshape(1, VOCAB))


_INFO = plsc.get_sparse_core_info()
_NC, _NS = _INFO.num_cores, _INFO.num_subcores
_NW = _NC * _NS

_ROWS = BATCH * SEQ
_R_PER_W = _ROWS // _NW
_CHUNK = 80
_N_CHUNKS = _R_PER_W // _CHUNK


@functools.partial(
    pl.kernel,
    mesh=plsc.VectorSubcoreMesh(core_axis_name="c", subcore_axis_name="s"),
    out_type=jax.ShapeDtypeStruct((_ROWS, VOCAB), jnp.float32),
    scratch_types=[
        pltpu.VMEM((_R_PER_W,), jnp.int32),
        pltpu.VMEM((_CHUNK, VOCAB), jnp.float32),
        pltpu.SemaphoreType.DMA,
    ],
    compiler_params=pltpu.CompilerParams(use_tc_tiling_on_sc=False),
)
def _gather(table_hbm, idx_hbm, out_hbm, idx_v, rows_v, sem):
    wid = lax.axis_index("s") * _NC + lax.axis_index("c")
    base = wid * _R_PER_W
    pltpu.sync_copy(idx_hbm.at[pl.ds(base, _R_PER_W)], idx_v)

    def step(i, carry):
        off = i * _CHUNK
        pltpu.async_copy(
            table_hbm.at[idx_v.at[pl.ds(off, _CHUNK)]], rows_v, sem
        ).wait()
        pltpu.sync_copy(rows_v, out_hbm.at[pl.ds(base + off, _CHUNK)])
        return carry

    lax.fori_loop(0, _N_CHUNKS, step, 0)


def kernel(input_ids, embed_weight, proj_weight, proj_bias):
    table = _build_table(embed_weight, proj_weight, proj_bias)
    ids = input_ids.reshape(-1).astype(jnp.int32)
    out = _gather(table, ids)
    return out.reshape(BATCH, SEQ, VOCAB)


# speedup vs baseline: 1.0045x; 1.0045x over previous
import functools

import jax
import jax.numpy as jnp
from jax import lax
from jax.experimental import pallas as pl
from jax.experimental.pallas import tpu as pltpu
from jax.experimental.pallas import tpu_sc as plsc

VOCAB = 1000
HIDDEN = 128
BATCH = 1024
SEQ = 50

def _table_body(embed_ref, proj_ref, bias_ref, table_ref):
    table_ref[...] = (
        lax.dot_general(
            embed_ref[...],
            proj_ref[...],
            dimension_numbers=(((1,), (1,)), ((), ())),
            preferred_element_type=jnp.float32,
        )
        + bias_ref[...]
    )


def _build_table(embed_weight, proj_weight, proj_bias):
    return pl.pallas_call(
        _table_body,
        out_shape=jax.ShapeDtypeStruct((VOCAB, VOCAB), jnp.float32),
    )(embed_weight, proj_weight, proj_bias.reshape(1, VOCAB))


_INFO = plsc.get_sparse_core_info()
_NC, _NS = _INFO.num_cores, _INFO.num_subcores
_NW = _NC * _NS

_ROWS = BATCH * SEQ
_R_PER_W = _ROWS // _NW
_CHUNK = 64
_N_CHUNKS = _R_PER_W // _CHUNK


@functools.partial(
    pl.kernel,
    mesh=plsc.VectorSubcoreMesh(core_axis_name="c", subcore_axis_name="s"),
    out_type=jax.ShapeDtypeStruct((_ROWS, VOCAB), jnp.float32),
    scratch_types=[
        pltpu.VMEM((_R_PER_W,), jnp.int32),
        pltpu.VMEM((_CHUNK, VOCAB), jnp.float32),
        pltpu.VMEM((_CHUNK, VOCAB), jnp.float32),
        pltpu.SemaphoreType.DMA,
        pltpu.SemaphoreType.DMA,
        pltpu.SemaphoreType.DMA,
        pltpu.SemaphoreType.DMA,
    ],
    compiler_params=pltpu.CompilerParams(use_tc_tiling_on_sc=False),
)
def _gather(table_hbm, idx_hbm, out_hbm, idx_v,
            buf0, buf1, gsem0, gsem1, ssem0, ssem1):
    wid = lax.axis_index("s") * _NC + lax.axis_index("c")
    base = wid * _R_PER_W
    pltpu.sync_copy(idx_hbm.at[pl.ds(base, _R_PER_W)], idx_v)

    bufs, gsems, ssems = (buf0, buf1), (gsem0, gsem1), (ssem0, ssem1)
    n = _N_CHUNKS

    def start_gather(i):
        b = i % 2
        return pltpu.async_copy(
            table_hbm.at[idx_v.at[pl.ds(i * _CHUNK, _CHUNK)]],
            bufs[b], gsems[b])

    def start_scatter(i):
        b = i % 2
        return pltpu.async_copy(
            bufs[b], out_hbm.at[pl.ds(base + i * _CHUNK, _CHUNK)], ssems[b])

    gathers = [None] * n
    scatters = [None] * n
    gathers[0] = start_gather(0)
    for i in range(n):
        gathers[i].wait()
        if i + 1 < n:
            if i >= 1:
                scatters[i - 1].wait()
            gathers[i + 1] = start_gather(i + 1)
        scatters[i] = start_scatter(i)
    scatters[n - 2].wait()
    scatters[n - 1].wait()


def kernel(input_ids, embed_weight, proj_weight, proj_bias):
    table = _build_table(embed_weight, proj_weight, proj_bias)
    ids = input_ids.reshape(-1).astype(jnp.int32)
    out = _gather(table, ids)
    return out.reshape(BATCH, SEQ, VOCAB)


# speedup vs baseline: 1.0069x; 1.0024x over previous
import functools

import jax
import jax.numpy as jnp
from jax import lax
from jax.experimental import pallas as pl
from jax.experimental.pallas import tpu as pltpu
from jax.experimental.pallas import tpu_sc as plsc

VOCAB = 1000
HIDDEN = 128
BATCH = 1024
SEQ = 50

def _table_body(embed_ref, proj_ref, bias_ref, table_ref):
    table_ref[...] = (
        lax.dot_general(
            embed_ref[...],
            proj_ref[...],
            dimension_numbers=(((1,), (1,)), ((), ())),
            preferred_element_type=jnp.float32,
        )
        + bias_ref[...]
    )


def _build_table(embed_weight, proj_weight, proj_bias):
    return pl.pallas_call(
        _table_body,
        out_shape=jax.ShapeDtypeStruct((VOCAB, VOCAB), jnp.float32),
    )(embed_weight, proj_weight, proj_bias.reshape(1, VOCAB))


_INFO = plsc.get_sparse_core_info()
_NC, _NS = _INFO.num_cores, _INFO.num_subcores
_NW = _NC * _NS

_B_PER_W = BATCH // _NW


@functools.partial(
    pl.kernel,
    mesh=plsc.VectorSubcoreMesh(core_axis_name="c", subcore_axis_name="s"),
    out_type=jax.ShapeDtypeStruct((BATCH, SEQ, VOCAB), jnp.float32),
    scratch_types=[
        pltpu.VMEM((_B_PER_W, SEQ), jnp.int32),
        pltpu.VMEM((SEQ, VOCAB), jnp.float32),
        pltpu.VMEM((SEQ, VOCAB), jnp.float32),
        pltpu.SemaphoreType.DMA,
        pltpu.SemaphoreType.DMA,
        pltpu.SemaphoreType.DMA,
        pltpu.SemaphoreType.DMA,
    ],
    compiler_params=pltpu.CompilerParams(use_tc_tiling_on_sc=False),
)
def _gather(table_hbm, idx_hbm, out_hbm, idx_v,
            buf0, buf1, gsem0, gsem1, ssem0, ssem1):
    wid = lax.axis_index("s") * _NC + lax.axis_index("c")
    base = wid * _B_PER_W
    pltpu.sync_copy(idx_hbm.at[pl.ds(base, _B_PER_W)], idx_v)

    bufs, gsems, ssems = (buf0, buf1), (gsem0, gsem1), (ssem0, ssem1)
    n = _B_PER_W

    def start_gather(i):
        b = i % 2
        return pltpu.async_copy(table_hbm.at[idx_v.at[i]], bufs[b], gsems[b])

    def start_scatter(i):
        b = i % 2
        return pltpu.async_copy(bufs[b], out_hbm.at[base + i], ssems[b])

    gathers = [None] * n
    scatters = [None] * n
    gathers[0] = start_gather(0)
    for i in range(n):
        gathers[i].wait()
        if i + 1 < n:
            if i >= 1:
                scatters[i - 1].wait()
            gathers[i + 1] = start_gather(i + 1)
        scatters[i] = start_scatter(i)
    scatters[n - 2].wait()
    scatters[n - 1].wait()


def kernel(input_ids, embed_weight, proj_weight, proj_bias):
    table = _build_table(embed_weight, proj_weight, proj_bias)
    ids = input_ids.astype(jnp.int32)
    return _gather(table, ids)


# speedup vs baseline: 1.4209x; 1.4112x over previous
import functools

import jax
import jax.numpy as jnp
from jax import lax
from jax.experimental import pallas as pl
from jax.experimental.pallas import tpu as pltpu
from jax.experimental.pallas import tpu_sc as plsc

VOCAB = 1000
HIDDEN = 128
BATCH = 1024
SEQ = 50

_VP = 1024

def _table_body(proj_ref, embed_ref, bias_ref, table_ref):
    table_ref[...] = (
        lax.dot_general(
            proj_ref[...],
            embed_ref[...],
            dimension_numbers=(((1,), (1,)), ((), ())),
            preferred_element_type=jnp.float32,
        )
        + bias_ref[...]
    )


def _build_table_t(embed_weight, proj_weight, proj_bias):
    proj_pad = jnp.pad(proj_weight, ((0, _VP - VOCAB), (0, 0)))
    embed_pad = jnp.pad(embed_weight, ((0, _VP - VOCAB), (0, 0)))
    bias_pad = jnp.pad(proj_bias, (0, _VP - VOCAB)).reshape(_VP, 1)
    return pl.pallas_call(
        _table_body,
        out_shape=jax.ShapeDtypeStruct((_VP, _VP), jnp.float32),
    )(proj_pad, embed_pad, bias_pad)


_INFO = plsc.get_sparse_core_info()
_NC, _NS = _INFO.num_cores, _INFO.num_subcores
_NW = _NC * _NS

_VT_PER_W = 4
_ROWS_W = _VT_PER_W * 8
_CHUNK_W = _ROWS_W * BATCH
_LAST_W = _NW - 1
_LAST_CHUNK = 8 * BATCH
_S_PLANE = VOCAB * BATCH
_OUT_WORDS = SEQ * _S_PLANE


@functools.partial(
    pl.kernel,
    mesh=plsc.VectorSubcoreMesh(core_axis_name="c", subcore_axis_name="s"),
    out_type=jax.ShapeDtypeStruct((_OUT_WORDS,), jnp.float32),
    scratch_types=[
        pltpu.VMEM((_ROWS_W, _VP), jnp.float32),
        pltpu.VMEM((BATCH,), jnp.int32),
        pltpu.VMEM((BATCH,), jnp.int32),
        pltpu.VMEM((_CHUNK_W,), jnp.float32),
        pltpu.VMEM((_CHUNK_W,), jnp.float32),
        pltpu.SemaphoreType.DMA,
        pltpu.SemaphoreType.DMA,
        pltpu.SemaphoreType.DMA,
        pltpu.SemaphoreType.DMA,
    ],
    compiler_params=pltpu.CompilerParams(
        use_tc_tiling_on_sc=False, needs_layout_passes=False),
)
def _gather_sc(table_hbm, ids_hbm, out_hbm,
               tbl_v, idx0, idx1, obuf0, obuf1,
               isem0, isem1, osem0, osem1):
    wid = lax.axis_index("s") * _NC + lax.axis_index("c")
    base_row = wid * _ROWS_W

    pltpu.sync_copy(table_hbm.at[pl.ds(base_row, _ROWS_W)], tbl_v)

    def start_idx(s, buf, sem):
        return pltpu.async_copy(ids_hbm.at[s], buf, sem)

    def wait_idx(buf, sem):
        pltpu.make_async_copy(ids_hbm.at[0], buf, sem).wait()

    def start_out(s, buf, sem):
        off = pl.multiple_of(s * _S_PLANE + wid * _CHUNK_W, 1024)

        @pl.when(wid < _LAST_W)
        def _():
            pltpu.async_copy(buf, out_hbm.at[pl.ds(off, _CHUNK_W)], sem)

        @pl.when(wid == _LAST_W)
        def _():
            pltpu.async_copy(
                buf.at[pl.ds(0, _LAST_CHUNK)],
                out_hbm.at[pl.ds(off, _LAST_CHUNK)], sem)

    def wait_out(buf, sem):
        @pl.when(wid < _LAST_W)
        def _():
            pltpu.make_async_copy(
                out_hbm.at[pl.ds(0, _CHUNK_W)], buf, sem).wait()

        @pl.when(wid == _LAST_W)
        def _():
            pltpu.make_async_copy(
                out_hbm.at[pl.ds(0, _LAST_CHUNK)],
                buf.at[pl.ds(0, _LAST_CHUNK)], sem).wait()

    def compute(idx_ref, out_ref):
        def bt_body(bt, carry):
            for bg in range(8):
                col = pl.multiple_of(bt * 128 + bg * 16, 16)
                ids16 = idx_ref[pl.ds(col, 16)]
                for vt in range(_VT_PER_W):
                    for vin in range(8):
                        vi = vt * 8 + vin
                        vals = plsc.load_gather(tbl_v.at[vi], [ids16])
                        off = pl.multiple_of(
                            vt * 8192 + bt * 1024 + vin * 128 + bg * 16, 16)
                        out_ref[pl.ds(off, 16)] = vals
            return carry
        lax.fori_loop(0, 8, bt_body, 0)

    start_idx(0, idx0, isem0)
    start_idx(1, idx1, isem1)

    def body(k, carry):
        s0 = 2 * k
        s1 = s0 + 1

        wait_idx(idx0, isem0)

        @pl.when(k > 0)
        def _():
            wait_out(obuf0, osem0)

        compute(idx0, obuf0)

        @pl.when(k < SEQ // 2 - 1)
        def _():
            start_idx(s0 + 2, idx0, isem0)

        start_out(s0, obuf0, osem0)

        wait_idx(idx1, isem1)

        @pl.when(k > 0)
        def _():
            wait_out(obuf1, osem1)

        compute(idx1, obuf1)

        @pl.when(k < SEQ // 2 - 1)
        def _():
            start_idx(s1 + 2, idx1, isem1)

        start_out(s1, obuf1, osem1)
        return carry

    lax.fori_loop(0, SEQ // 2, body, 0)
    wait_out(obuf0, osem0)
    wait_out(obuf1, osem1)


def kernel(input_ids, embed_weight, proj_weight, proj_bias):
    table_t = _build_table_t(embed_weight, proj_weight, proj_bias)
    ids_t = input_ids.T.astype(jnp.int32)
    out1d = _gather_sc(table_t, ids_t)
    out5 = out1d.reshape(SEQ, VOCAB // 8, 8, 8, 128)
    return out5.transpose(2, 4, 0, 1, 3).reshape(BATCH, SEQ, VOCAB)


# speedup vs baseline: 2.8314x; 1.9926x over previous
import functools

import jax
import jax.numpy as jnp
from jax import lax
from jax.experimental import pallas as pl
from jax.experimental.pallas import tpu as pltpu
from jax.experimental.pallas import tpu_sc as plsc

VOCAB = 1000
HIDDEN = 128
BATCH = 1024
SEQ = 50

_VP = 1024

def _table_body(proj_ref, embed_ref, bias_ref, table_ref):
    table_ref[...] = (
        lax.dot_general(
            proj_ref[...],
            embed_ref[...],
            dimension_numbers=(((1,), (1,)), ((), ())),
            preferred_element_type=jnp.float32,
        )
        + bias_ref[...]
    )


def _build_table_t(embed_weight, proj_weight, proj_bias):
    proj_pad = jnp.pad(proj_weight, ((0, _VP - VOCAB), (0, 0)))
    embed_pad = jnp.pad(embed_weight, ((0, _VP - VOCAB), (0, 0)))
    bias_pad = jnp.pad(proj_bias, (0, _VP - VOCAB)).reshape(_VP, 1)
    return pl.pallas_call(
        _table_body,
        out_shape=jax.ShapeDtypeStruct((_VP, _VP), jnp.float32),
    )(proj_pad, embed_pad, bias_pad)


_INFO = plsc.get_sparse_core_info()
_NC, _NS = _INFO.num_cores, _INFO.num_subcores
_NW = _NC * _NS

_VT_PER_W = 4
_ROWS_W = _VT_PER_W * 8
_CHUNK_W = _ROWS_W * BATCH
_LAST_W = _NW - 1
_LAST_CHUNK = 8 * BATCH
_S_PLANE = VOCAB * BATCH
_OUT_WORDS = SEQ * _S_PLANE


@functools.partial(
    pl.kernel,
    mesh=plsc.VectorSubcoreMesh(core_axis_name="c", subcore_axis_name="s"),
    out_type=jax.ShapeDtypeStruct((_OUT_WORDS,), jnp.float32),
    scratch_types=[
        pltpu.VMEM((_ROWS_W, _VP), jnp.float32),
        pltpu.VMEM((BATCH,), jnp.int32),
        pltpu.VMEM((BATCH,), jnp.int32),
        pltpu.VMEM((_CHUNK_W,), jnp.float32),
        pltpu.VMEM((_CHUNK_W,), jnp.float32),
        pltpu.SemaphoreType.DMA,
        pltpu.SemaphoreType.DMA,
        pltpu.SemaphoreType.DMA,
        pltpu.SemaphoreType.DMA,
    ],
    compiler_params=pltpu.CompilerParams(
        use_tc_tiling_on_sc=False, needs_layout_passes=False),
)
def _gather_sc(table_hbm, ids_hbm, out_hbm,
               tbl_v, idx0, idx1, obuf0, obuf1,
               isem0, isem1, osem0, osem1):
    wid = lax.axis_index("s") * _NC + lax.axis_index("c")
    base_row = wid * _ROWS_W

    pltpu.sync_copy(table_hbm.at[pl.ds(base_row, _ROWS_W)], tbl_v)

    def start_idx(s, buf, sem):
        return pltpu.async_copy(ids_hbm.at[s], buf, sem)

    def wait_idx(buf, sem):
        pltpu.make_async_copy(ids_hbm.at[0], buf, sem).wait()

    def start_out(s, buf, sem):
        off = pl.multiple_of(s * _S_PLANE + wid * _CHUNK_W, 1024)

        @pl.when(wid < _LAST_W)
        def _():
            pltpu.async_copy(buf, out_hbm.at[pl.ds(off, _CHUNK_W)], sem)

        @pl.when(wid == _LAST_W)
        def _():
            pltpu.async_copy(
                buf.at[pl.ds(0, _LAST_CHUNK)],
                out_hbm.at[pl.ds(off, _LAST_CHUNK)], sem)

    def wait_out(buf, sem):
        @pl.when(wid < _LAST_W)
        def _():
            pltpu.make_async_copy(
                out_hbm.at[pl.ds(0, _CHUNK_W)], buf, sem).wait()

        @pl.when(wid == _LAST_W)
        def _():
            pltpu.make_async_copy(
                out_hbm.at[pl.ds(0, _LAST_CHUNK)],
                buf.at[pl.ds(0, _LAST_CHUNK)], sem).wait()

    def compute(idx_ref, out_ref):
        def bt_body(bt, carry):
            ids = [
                idx_ref[pl.ds(pl.multiple_of(bt * 128 + bg * 16, 16), 16)]
                for bg in range(8)
            ]
            for bg in range(8):
                vals = [
                    plsc.load_gather(tbl_v.at[vi], [ids[bg]])
                    for vi in range(_ROWS_W)
                ]
                for vi in range(_ROWS_W):
                    off = pl.multiple_of(
                        (vi // 8) * 8192 + bt * 1024
                        + (vi % 8) * 128 + bg * 16, 16)
                    out_ref[pl.ds(off, 16)] = vals[vi]
            return carry
        lax.fori_loop(0, 8, bt_body, 0)

    start_idx(0, idx0, isem0)
    start_idx(1, idx1, isem1)

    def body(k, carry):
        s0 = 2 * k
        s1 = s0 + 1

        wait_idx(idx0, isem0)

        @pl.when(k > 0)
        def _():
            wait_out(obuf0, osem0)

        compute(idx0, obuf0)

        @pl.when(k < SEQ // 2 - 1)
        def _():
            start_idx(s0 + 2, idx0, isem0)

        start_out(s0, obuf0, osem0)

        wait_idx(idx1, isem1)

        @pl.when(k > 0)
        def _():
            wait_out(obuf1, osem1)

        compute(idx1, obuf1)

        @pl.when(k < SEQ // 2 - 1)
        def _():
            start_idx(s1 + 2, idx1, isem1)

        start_out(s1, obuf1, osem1)
        return carry

    lax.fori_loop(0, SEQ // 2, body, 0)
    wait_out(obuf0, osem0)
    wait_out(obuf1, osem1)


def kernel(input_ids, embed_weight, proj_weight, proj_bias):
    table_t = _build_table_t(embed_weight, proj_weight, proj_bias)
    ids_t = input_ids.T.astype(jnp.int32)
    out1d = _gather_sc(table_t, ids_t)
    out5 = out1d.reshape(SEQ, VOCAB // 8, 8, 8, 128)
    return out5.transpose(2, 4, 0, 1, 3).reshape(BATCH, SEQ, VOCAB)


# speedup vs baseline: 3.8649x; 1.3650x over previous
import functools

import jax
import jax.numpy as jnp
from jax import lax
from jax.experimental import pallas as pl
from jax.experimental.pallas import tpu as pltpu
from jax.experimental.pallas import tpu_sc as plsc

VOCAB = 1000
HIDDEN = 128
BATCH = 1024
SEQ = 50

_VP = 1024
_NPAIR = _VP // 2
_NJT = 8

def _packed_body(pe_ref, po_ref, eb_ref, be_ref, bo_ref, out_ref):
    def half(p_ref, b_ref):
        t = lax.dot_general(
            p_ref[...], eb_ref[...],
            dimension_numbers=(((1,), (1,)), ((), ())),
            preferred_element_type=jnp.float32,
        ) + b_ref[...]
        u16 = lax.bitcast_convert_type(t.astype(jnp.bfloat16), jnp.uint16)
        return u16.astype(jnp.uint32)

    ue = half(pe_ref, be_ref)
    uo = half(po_ref, bo_ref)
    out_ref[...] = lax.bitcast_convert_type(ue | (uo << 16), jnp.int32)


def _build_packed_table(embed_weight, proj_weight, proj_bias):
    pe = jnp.pad(proj_weight[0::2], ((0, _NPAIR - 500), (0, 0)))
    po = jnp.pad(proj_weight[1::2], ((0, _NPAIR - 500), (0, 0)))
    eb = jnp.pad(embed_weight, ((0, _VP - VOCAB), (0, 0)))
    be = jnp.pad(proj_bias[0::2], (0, _NPAIR - 500)).reshape(_NPAIR, 1)
    bo = jnp.pad(proj_bias[1::2], (0, _NPAIR - 500)).reshape(_NPAIR, 1)
    x = pl.pallas_call(
        _packed_body,
        grid=(_NJT,),
        in_specs=[
            pl.BlockSpec((_NPAIR, HIDDEN), lambda jt: (0, 0)),
            pl.BlockSpec((_NPAIR, HIDDEN), lambda jt: (0, 0)),
            pl.BlockSpec((128, HIDDEN), lambda jt: (jt, 0)),
            pl.BlockSpec((_NPAIR, 1), lambda jt: (0, 0)),
            pl.BlockSpec((_NPAIR, 1), lambda jt: (0, 0)),
        ],
        out_specs=pl.BlockSpec((_NPAIR, 128), lambda jt: (jt, 0)),
        out_shape=jax.ShapeDtypeStruct((_NJT * _NPAIR, 128), jnp.int32),
    )(pe, po, eb, be, bo)
    return x.reshape(-1)


_INFO = plsc.get_sparse_core_info()
_NC, _NS = _INFO.num_cores, _INFO.num_subcores
_NW = _NC * _NS

_VT_PER_W = 4
_ROWS_W = _VT_PER_W * 8
_PAIRS_W = _ROWS_W // 2
_TBL_W = _NJT * _PAIRS_W * 128
_IDX_SPAN = (_NJT - 1) * _PAIRS_W * 128 + 128
_CHUNK_W = _ROWS_W * BATCH
_LAST_W = _NW - 1
_LAST_CHUNK = 8 * BATCH
_S_PLANE = VOCAB * BATCH
_OUT_WORDS = SEQ * _S_PLANE


@functools.partial(
    pl.kernel,
    mesh=plsc.VectorSubcoreMesh(core_axis_name="c", subcore_axis_name="s"),
    out_type=jax.ShapeDtypeStruct((_OUT_WORDS,), jnp.float32),
    scratch_types=[
        pltpu.VMEM((_TBL_W,), jnp.int32),
        pltpu.VMEM((BATCH,), jnp.int32),
        pltpu.VMEM((BATCH,), jnp.int32),
        pltpu.VMEM((_CHUNK_W,), jnp.float32),
        pltpu.VMEM((_CHUNK_W,), jnp.float32),
        pltpu.SemaphoreType.DMA,
        pltpu.SemaphoreType.DMA,
        pltpu.SemaphoreType.DMA,
        pltpu.SemaphoreType.DMA,
    ],
    compiler_params=pltpu.CompilerParams(
        use_tc_tiling_on_sc=False, needs_layout_passes=False),
)
def _gather_sc(table_hbm, ids_hbm, out_hbm,
               tbl_v, idx0, idx1, obuf0, obuf1,
               isem0, isem1, osem0, osem1):
    wid = lax.axis_index("s") * _NC + lax.axis_index("c")

    for jt in range(_NJT):
        pltpu.sync_copy(
            table_hbm.at[pl.ds(jt * _NPAIR * 128 + wid * _PAIRS_W * 128,
                               _PAIRS_W * 128)],
            tbl_v.at[pl.ds(jt * _PAIRS_W * 128, _PAIRS_W * 128)])

    def start_idx(s, buf, sem):
        return pltpu.async_copy(ids_hbm.at[s], buf, sem)

    def wait_idx(buf, sem):
        pltpu.make_async_copy(ids_hbm.at[0], buf, sem).wait()

    def start_out(s, buf, sem):
        off = pl.multiple_of(s * _S_PLANE + wid * _CHUNK_W, 1024)

        @pl.when(wid < _LAST_W)
        def _():
            pltpu.async_copy(buf, out_hbm.at[pl.ds(off, _CHUNK_W)], sem)

        @pl.when(wid == _LAST_W)
        def _():
            pltpu.async_copy(
                buf.at[pl.ds(0, _LAST_CHUNK)],
                out_hbm.at[pl.ds(off, _LAST_CHUNK)], sem)

    def wait_out(buf, sem):
        @pl.when(wid < _LAST_W)
        def _():
            pltpu.make_async_copy(
                out_hbm.at[pl.ds(0, _CHUNK_W)], buf, sem).wait()

        @pl.when(wid == _LAST_W)
        def _():
            pltpu.make_async_copy(
                out_hbm.at[pl.ds(0, _LAST_CHUNK)],
                buf.at[pl.ds(0, _LAST_CHUNK)], sem).wait()

    def compute(idx_ref, out_ref):
        def bt_body(bt, carry):
            ids = [
                idx_ref[pl.ds(pl.multiple_of(bt * 128 + bg * 16, 16), 16)]
                for bg in range(8)
            ]
            for bg in range(8):
                gs = [
                    plsc.load_gather(
                        tbl_v.at[pl.ds(vp * 128, _IDX_SPAN)], [ids[bg]])
                    for vp in range(_PAIRS_W)
                ]
                for vp in range(_PAIRS_W):
                    lo, hi = plsc.unpack(
                        plsc.bitcast(gs[vp], jnp.bfloat16),
                        format=plsc.PackFormat.INTERLEAVED,
                        preferred_element_type=jnp.float32)
                    for phase, vals in ((0, lo), (1, hi)):
                        vl = 2 * vp + phase
                        off = pl.multiple_of(
                            (vl // 8) * 8192 + bt * 1024
                            + (vl % 8) * 128 + bg * 16, 16)
                        out_ref[pl.ds(off, 16)] = vals
            return carry
        lax.fori_loop(0, 8, bt_body, 0)

    start_idx(0, idx0, isem0)
    start_idx(1, idx1, isem1)

    def body(k, carry):
        s0 = 2 * k
        s1 = s0 + 1

        wait_idx(idx0, isem0)

        @pl.when(k > 0)
        def _():
            wait_out(obuf0, osem0)

        compute(idx0, obuf0)

        @pl.when(k < SEQ // 2 - 1)
        def _():
            start_idx(s0 + 2, idx0, isem0)

        start_out(s0, obuf0, osem0)

        wait_idx(idx1, isem1)

        @pl.when(k > 0)
        def _():
            wait_out(obuf1, osem1)

        compute(idx1, obuf1)

        @pl.when(k < SEQ // 2 - 1)
        def _():
            start_idx(s1 + 2, idx1, isem1)

        start_out(s1, obuf1, osem1)
        return carry

    lax.fori_loop(0, SEQ // 2, body, 0)
    wait_out(obuf0, osem0)
    wait_out(obuf1, osem1)


def kernel(input_ids, embed_weight, proj_weight, proj_bias):
    table = _build_packed_table(embed_weight, proj_weight, proj_bias)
    ids = input_ids.T.astype(jnp.int32)
    tids = ((ids >> 7) << 11) | (ids & 127)
    out1d = _gather_sc(table, tids)
    out5 = out1d.reshape(SEQ, VOCAB // 8, 8, 8, 128)
    return out5.transpose(2, 4, 0, 1, 3).reshape(BATCH, SEQ, VOCAB)


# speedup vs baseline: 4.1514x; 1.0741x over previous
import functools

import jax
import jax.numpy as jnp
from jax import lax
from jax.experimental import pallas as pl
from jax.experimental.pallas import tpu as pltpu
from jax.experimental.pallas import tpu_sc as plsc

VOCAB = 1000
HIDDEN = 128
BATCH = 1024
SEQ = 50

_VP = 1024
_NPAIR = _VP // 2
_NJT = 8

def _packed_body(pe_ref, po_ref, eb_ref, be_ref, bo_ref, out_ref):
    def half(p_ref, b_ref):
        t = lax.dot_general(
            p_ref[...], eb_ref[...],
            dimension_numbers=(((1,), (1,)), ((), ())),
            preferred_element_type=jnp.float32,
        ) + b_ref[...]
        u16 = lax.bitcast_convert_type(t.astype(jnp.bfloat16), jnp.uint16)
        return u16.astype(jnp.uint32)

    ue = half(pe_ref, be_ref)
    uo = half(po_ref, bo_ref)
    out_ref[...] = lax.bitcast_convert_type(ue | (uo << 16), jnp.int32)


def _build_packed_table(embed_weight, proj_weight, proj_bias):
    pe = jnp.pad(proj_weight[0::2], ((0, _NPAIR - 500), (0, 0)))
    po = jnp.pad(proj_weight[1::2], ((0, _NPAIR - 500), (0, 0)))
    eb = jnp.pad(embed_weight, ((0, _VP - VOCAB), (0, 0)))
    be = jnp.pad(proj_bias[0::2], (0, _NPAIR - 500)).reshape(_NPAIR, 1)
    bo = jnp.pad(proj_bias[1::2], (0, _NPAIR - 500)).reshape(_NPAIR, 1)
    x = pl.pallas_call(
        _packed_body,
        grid=(_NJT,),
        in_specs=[
            pl.BlockSpec((_NPAIR, HIDDEN), lambda jt: (0, 0)),
            pl.BlockSpec((_NPAIR, HIDDEN), lambda jt: (0, 0)),
            pl.BlockSpec((128, HIDDEN), lambda jt: (jt, 0)),
            pl.BlockSpec((_NPAIR, 1), lambda jt: (0, 0)),
            pl.BlockSpec((_NPAIR, 1), lambda jt: (0, 0)),
        ],
        out_specs=pl.BlockSpec((_NPAIR, 128), lambda jt: (jt, 0)),
        out_shape=jax.ShapeDtypeStruct((_NJT * _NPAIR, 128), jnp.int32),
    )(pe, po, eb, be, bo)
    return x.reshape(-1)


_INFO = plsc.get_sparse_core_info()
_NC, _NS = _INFO.num_cores, _INFO.num_subcores
_NW = _NC * _NS

_VT_PER_W = 4
_ROWS_W = _VT_PER_W * 8
_PAIRS_W = _ROWS_W // 2
_TBL_W = _NJT * _PAIRS_W * 128
_IDX_SPAN = (_NJT - 1) * _PAIRS_W * 128 + 128
_CHUNK_W = _ROWS_W * BATCH
_LAST_W = _NW - 1
_LAST_CHUNK = 8 * BATCH
_S_PLANE = VOCAB * BATCH
_OUT_WORDS = SEQ * _S_PLANE


@functools.partial(
    pl.kernel,
    mesh=plsc.VectorSubcoreMesh(core_axis_name="c", subcore_axis_name="s"),
    out_type=jax.ShapeDtypeStruct((_OUT_WORDS,), jnp.float32),
    scratch_types=[
        pltpu.VMEM((_TBL_W,), jnp.int32),
        pltpu.VMEM((BATCH,), jnp.int32),
        pltpu.VMEM((BATCH,), jnp.int32),
        pltpu.VMEM((_CHUNK_W,), jnp.float32),
        pltpu.VMEM((_CHUNK_W,), jnp.float32),
        pltpu.SemaphoreType.DMA,
        pltpu.SemaphoreType.DMA,
        pltpu.SemaphoreType.DMA,
        pltpu.SemaphoreType.DMA,
    ],
    compiler_params=pltpu.CompilerParams(
        use_tc_tiling_on_sc=False, needs_layout_passes=False),
)
def _gather_sc(table_hbm, ids_hbm, out_hbm,
               tbl_v, idx0, idx1, obuf0, obuf1,
               isem0, isem1, osem0, osem1):
    wid = lax.axis_index("s") * _NC + lax.axis_index("c")

    for jt in range(_NJT):
        pltpu.sync_copy(
            table_hbm.at[pl.ds(jt * _NPAIR * 128 + wid * _PAIRS_W * 128,
                               _PAIRS_W * 128)],
            tbl_v.at[pl.ds(jt * _PAIRS_W * 128, _PAIRS_W * 128)])

    def start_idx(s, buf, sem):
        return pltpu.async_copy(ids_hbm.at[s], buf, sem)

    def wait_idx(buf, sem):
        pltpu.make_async_copy(ids_hbm.at[0], buf, sem).wait()

    def start_out(s, buf, sem):
        off = pl.multiple_of(s * _S_PLANE + wid * _CHUNK_W, 1024)

        @pl.when(wid < _LAST_W)
        def _():
            pltpu.async_copy(buf, out_hbm.at[pl.ds(off, _CHUNK_W)], sem)

        @pl.when(wid == _LAST_W)
        def _():
            pltpu.async_copy(
                buf.at[pl.ds(0, _LAST_CHUNK)],
                out_hbm.at[pl.ds(off, _LAST_CHUNK)], sem)

    def wait_out(buf, sem):
        @pl.when(wid < _LAST_W)
        def _():
            pltpu.make_async_copy(
                out_hbm.at[pl.ds(0, _CHUNK_W)], buf, sem).wait()

        @pl.when(wid == _LAST_W)
        def _():
            pltpu.make_async_copy(
                out_hbm.at[pl.ds(0, _LAST_CHUNK)],
                buf.at[pl.ds(0, _LAST_CHUNK)], sem).wait()

    def compute(idx_ref, out_ref):
        def bt_body(bt, carry):
            ids = [
                idx_ref[pl.ds(pl.multiple_of(bt * 128 + bg * 16, 16), 16)]
                for bg in range(8)
            ]
            for bg in range(8):
                for vp0 in range(0, _PAIRS_W, 8):
                    gs = [
                        plsc.load_gather(
                            tbl_v.at[pl.ds(vp * 128, _IDX_SPAN)], [ids[bg]])
                        for vp in range(vp0, vp0 + 8)
                    ]
                    for i, vp in enumerate(range(vp0, vp0 + 8)):
                        lo, hi = plsc.unpack(
                            plsc.bitcast(gs[i], jnp.bfloat16),
                            format=plsc.PackFormat.INTERLEAVED,
                            preferred_element_type=jnp.float32)
                        for phase, vals in ((0, lo), (1, hi)):
                            vl = 2 * vp + phase
                            off = pl.multiple_of(
                                (vl // 8) * 8192 + bt * 1024
                                + (vl % 8) * 128 + bg * 16, 16)
                            out_ref[pl.ds(off, 16)] = vals
            return carry
        lax.fori_loop(0, 8, bt_body, 0)

    start_idx(0, idx0, isem0)
    start_idx(1, idx1, isem1)

    def body(k, carry):
        s0 = 2 * k
        s1 = s0 + 1

        wait_idx(idx0, isem0)

        @pl.when(k > 0)
        def _():
            wait_out(obuf0, osem0)

        compute(idx0, obuf0)

        @pl.when(k < SEQ // 2 - 1)
        def _():
            start_idx(s0 + 2, idx0, isem0)

        start_out(s0, obuf0, osem0)

        wait_idx(idx1, isem1)

        @pl.when(k > 0)
        def _():
            wait_out(obuf1, osem1)

        compute(idx1, obuf1)

        @pl.when(k < SEQ // 2 - 1)
        def _():
            start_idx(s1 + 2, idx1, isem1)

        start_out(s1, obuf1, osem1)
        return carry

    lax.fori_loop(0, SEQ // 2, body, 0)
    wait_out(obuf0, osem0)
    wait_out(obuf1, osem1)


def kernel(input_ids, embed_weight, proj_weight, proj_bias):
    table = _build_packed_table(embed_weight, proj_weight, proj_bias)
    ids = input_ids.T.astype(jnp.int32)
    tids = ((ids >> 7) << 11) | (ids & 127)
    out1d = _gather_sc(table, tids)
    out5 = out1d.reshape(SEQ, VOCAB // 8, 8, 8, 128)
    return out5.transpose(2, 4, 0, 1, 3).reshape(BATCH, SEQ, VOCAB)
